# baseline (speedup 1.0000x reference)
import functools

import jax
import jax.numpy as jnp
from jax import lax
from jax.experimental import pallas as pl
from jax.experimental.pallas import tpu as pltpu
from jax.experimental.pallas import tpu_sc as plsc

C = 128
K = 15
CPG = 16
GROUPS = 8
SIGMA = 2.0
EPS = 1e-5

NC = 2
NS = 16
NW = NC * NS
CHQ = 4
KCPG = K * CPG
CHALF = C // 2


def _make_sc_gather_pts(n_chunks_total, n_chunks_w, m_tab):
    mesh = plsc.VectorSubcoreMesh(core_axis_name="c", subcore_axis_name="s")

    @functools.partial(
        pl.kernel,
        mesh=mesh,
        out_type=[jax.ShapeDtypeStruct((n_chunks_total, 128), jnp.float32)
                  for _ in range(3)],
        compiler_params=pltpu.CompilerParams(
            use_tc_tiling_on_sc=False, needs_layout_passes=False),
        scratch_types=[
            pltpu.VMEM((m_tab, 8), jnp.float32),
            pltpu.VMEM((n_chunks_w, 128), jnp.int32),
            pltpu.VMEM((CHQ * n_chunks_w, 16), jnp.float32),
            pltpu.VMEM((3, 128), jnp.float32),
            pltpu.VMEM((3, 128), jnp.float32),
            pltpu.SemaphoreType.DMA,
            pltpu.SemaphoreType.DMA,
        ],
    )
    def sc_gather_pts(spts_hbm, idx_hbm, qp_hbm, x_hbm, y_hbm, z_hbm,
                      spts_v, idx_f, qp_f, pts0, pts1, so0, so1):
        wid = lax.axis_index("s") * NC + lax.axis_index("c")
        pts = (pts0, pts1)
        so = (so0, so1)
        planes = (x_hbm, y_hbm, z_hbm)

        pltpu.sync_copy(spts_hbm, spts_v)
        pltpu.sync_copy(idx_hbm.at[pl.ds(wid * n_chunks_w, n_chunks_w)], idx_f)
        pltpu.sync_copy(
            qp_hbm.at[pl.ds(wid * n_chunks_w * CHQ, n_chunks_w * CHQ)], qp_f)

        def start_out(c, p):
            gc = wid * n_chunks_w + c
            for coord in range(3):
                pltpu.async_copy(pts[p].at[coord], planes[coord].at[gc], so[p])

        def wait_out(c, p):
            gc = wid * n_chunks_w + c
            for coord in range(3):
                pltpu.make_async_copy(
                    pts[p].at[coord], planes[coord].at[gc], so[p]).wait()

        def body(c2, carry):
            for p in range(2):
                c = c2 * 2 + p

                @pl.when(c >= 2)
                def _():
                    wait_out(c, p)

                for j in range(8):
                    qrow = qp_f[c * CHQ + j // 2, :]
                    idx16 = idx_f[c, pl.ds(j * 16, 16)]
                    for coord in range(3):
                        cidx = jnp.full((16,), coord, jnp.int32)
                        v = plsc.load_gather(spts_v, [idx16, cidx])
                        pts[p][coord, pl.ds(j * 16, 16)] = v - qrow[coord]
                start_out(c, p)
            return carry

        lax.fori_loop(0, n_chunks_w // 2, body, 0)
        for p in range(2):
            wait_out(n_chunks_w - 2 + p, p)

    return sc_gather_pts


def _mlp_body(m_real, n_blocks, bm, sf_ref, w1_ref, b1_ref, w2_ref, b2_ref,
              cw_ref, muinv_ref, acc_ref):
    i = pl.program_id(0)
    h = jnp.dot(sf_ref[...], w1_ref[...], preferred_element_type=jnp.float32)
    h = h + b1_ref[...]
    h = jnp.where(h >= 0.0, h, 0.1 * h)
    cw = jnp.dot(h, w2_ref[...], preferred_element_type=jnp.float32)
    cw = cw + b2_ref[...]
    cw_ref[...] = cw

    row = i * bm + lax.broadcasted_iota(jnp.int32, (bm, 1), 0)
    mask = row < m_real
    cw_m = jnp.where(mask, cw, 0.0)
    s1 = jnp.sum(cw_m, axis=0, keepdims=True)
    s2 = jnp.sum(cw_m * cw_m, axis=0, keepdims=True)

    @pl.when(i == 0)
    def _():
        acc_ref[...] = jnp.zeros_like(acc_ref)

    acc_ref[0:1, :] += s1
    acc_ref[1:2, :] += s2

    @pl.when(i == n_blocks - 1)
    def _():
        inv_n = 1.0 / (CPG * m_real)
        for k in range(K):
            sl = slice(k * CPG, (k + 1) * CPG)
            gs1 = jnp.sum(acc_ref[0:1, sl])
            gs2 = jnp.sum(acc_ref[1:2, sl])
            mu = gs1 * inv_n
            var = gs2 * inv_n - mu * mu
            inv_sd = lax.rsqrt(var + EPS)
            muinv_ref[0:1, sl] = jnp.full((1, CPG), mu, jnp.float32)
            muinv_ref[1:2, sl] = jnp.full((1, CPG), inv_sd, jnp.float32)


def _weights_body(cb, cw_ref, muinv_ref, gam_ref, bet_ref,
                  x_ref, y_ref, z_ref, kp_ref, cwn_ref, bestk_ref, infl_ref):
    x = x_ref[...]
    y = y_ref[...]
    z = z_ref[...]

    base = x * x + y * y + z * z
    best = jnp.full((cb, 128), 1e30, dtype=jnp.float32)
    bestk = jnp.zeros((cb, 128), dtype=jnp.int32)
    for k in range(K):
        kx = kp_ref[k : k + 1, 0:1]
        ky = kp_ref[k : k + 1, 1:2]
        kz = kp_ref[k : k + 1, 2:3]
        c_k = kx * kx + ky * ky + kz * kz
        sq = c_k - 2.0 * (x * kx + y * ky + z * kz)
        better = sq < best
        best = jnp.where(better, sq, best)
        bestk = jnp.where(better, k, bestk)

    nn_sq = jnp.maximum(best + base, 0.0)
    infl_ref[...] = jnp.maximum(1.0 - jnp.sqrt(nn_sq) / SIGMA, 0.0)
    bestk_ref[...] = bestk
    cwn_ref[...] = ((cw_ref[...] - muinv_ref[0:1, :]) * muinv_ref[1:2, :]
                    * gam_ref[...] + bet_ref[...])


def _make_sc_aggregate(mp, n_pair, m_tab):
    mesh = plsc.VectorSubcoreMesh(core_axis_name="c", subcore_axis_name="s")

    n_w = n_pair // 2
    n_s = n_w // 2
    SQ = 2 * CHQ

    @functools.partial(
        pl.kernel,
        mesh=mesh,
        out_type=jax.ShapeDtypeStruct((mp, C), jnp.float32),
        compiler_params=pltpu.CompilerParams(
            use_tc_tiling_on_sc=False, needs_layout_passes=False),
        scratch_types=[
            pltpu.VMEM_SHARED((m_tab, C // 2), jnp.int32),
            pltpu.VMEM((n_w, 128), jnp.int32),
            pltpu.VMEM((n_w, 128), jnp.int32),
            pltpu.VMEM((n_w, 128), jnp.float32),
            pltpu.VMEM((SQ, KCPG), jnp.float32),
            pltpu.VMEM((SQ, KCPG), jnp.float32),
            pltpu.VMEM((256 * CPG,), jnp.float32),
            pltpu.VMEM((256, C // 2), jnp.int32),
            pltpu.VMEM((256, C // 2), jnp.int32),
            pltpu.VMEM((SQ, C), jnp.float32),
            pltpu.VMEM((SQ, C), jnp.float32),
            pltpu.SemaphoreType.DMA,
            pltpu.SemaphoreType.DMA,
            pltpu.SemaphoreType.DMA,
            pltpu.SemaphoreType.DMA,
            pltpu.SemaphoreType.DMA,
            pltpu.SemaphoreType.DMA,
        ],
    )
    def sc_aggregate(feats_hbm, idx_hbm, bk_hbm, infl_hbm, cwn_hbm, out_hbm,
                     feats_spm, idx_f, bk_f, in_f, cw0, cw1, wf_v, rows0, rows1,
                     out0, out1, sg0, sg1, sc0, sc1, so0, so1):
        core = lax.axis_index("c")
        tile = lax.axis_index("s")
        wid = tile * NC + core

        @pl.when(tile == 0)
        def _():
            pltpu.sync_copy(feats_hbm, feats_spm)

        plsc.subcore_barrier()
        lane16 = jnp.arange(16, dtype=jnp.int32) * CPG
        rows = (rows0, rows1)
        cw = (cw0, cw1)
        outv = (out0, out1)
        sg = (sg0, sg1)
        sc = (sc0, sc1)
        so = (so0, so1)

        base = wid * n_w

        pltpu.sync_copy(idx_hbm.at[pl.ds(base, n_w)], idx_f)
        pltpu.sync_copy(bk_hbm.at[pl.ds(base, n_w)], bk_f)
        pltpu.sync_copy(infl_hbm.at[pl.ds(base, n_w)], in_f)

        def start_g(s, p):
            pltpu.async_copy(feats_spm.at[idx_f.at[2 * s]],
                             rows[p].at[pl.ds(0, 128)], sg[p])
            pltpu.async_copy(feats_spm.at[idx_f.at[2 * s + 1]],
                             rows[p].at[pl.ds(128, 128)], sg[p])
            pltpu.async_copy(cwn_hbm.at[pl.ds((base + 2 * s) * CHQ, SQ)],
                             cw[p], sc[p])

        def wait_g(s, p):
            pltpu.make_async_copy(feats_spm.at[idx_f.at[2 * s]],
                                  rows[p].at[pl.ds(0, 128)], sg[p]).wait()
            pltpu.make_async_copy(feats_spm.at[idx_f.at[2 * s + 1]],
                                  rows[p].at[pl.ds(128, 128)], sg[p]).wait()
            pltpu.make_async_copy(cwn_hbm.at[pl.ds((base + 2 * s) * CHQ, SQ)],
                                  cw[p], sc[p]).wait()

        def out_dst(s):
            return out_hbm.at[pl.ds((base + 2 * s) * CHQ, SQ)]

        start_g(0, 0)
        start_g(1, 1)

        def body(s2, carry):
            for p in range(2):
                s = s2 * 2 + p
                wait_g(s, p)

                @pl.when(s >= 2)
                def _():
                    pltpu.make_async_copy(outv[p], out_dst(s), so[p]).wait()

                def wbody(j2, carry2):
                    for dj in range(2):
                        j = j2 * 2 + dj
                        r = 2 * s + j // 8
                        sl = (j % 8) * 16
                        k16 = bk_f[r, pl.ds(sl, 16)]
                        f16 = in_f[r, pl.ds(sl, 16)]
                        q16 = jnp.full((16,), 0, jnp.int32) + j // 2
                        src = k16 * CPG
                        dst = j * 256 + lane16
                        for cc in range(CPG):
                            wc = plsc.load_gather(cw[p], [q16, src + cc]) * f16
                            plsc.store_scatter(wf_v, [dst + cc], wc)
                    return carry2

                lax.fori_loop(0, 8, wbody, 0)

                for q in range(SQ):
                    def hbody(h2, accs):
                        res = list(accs)
                        for dh in range(4):
                            e = q * 32 + h2 * 4 + dh
                            w = wf_v[pl.ds(e * CPG, CPG)]
                            for L in range(4):
                                v = rows[p][e, pl.ds(L * 16, 16)]
                                vb = plsc.bitcast(v, jnp.bfloat16)
                                a, b = plsc.unpack(
                                    vb, format=plsc.PackFormat.INTERLEAVED)
                                res[2 * L] = res[2 * L] + a * w
                                res[2 * L + 1] = res[2 * L + 1] + b * w
                        return tuple(res)
                    accs = lax.fori_loop(
                        0, 8, hbody,
                        tuple(jnp.zeros((CPG,), jnp.float32)
                              for _ in range(GROUPS)),
                    )
                    for g in range(GROUPS):
                        outv[p][q, pl.ds(g * CPG, CPG)] = accs[g]

                @pl.when(s + 2 < n_s)
                def _():
                    start_g(s + 2, p)

                pltpu.async_copy(outv[p], out_dst(s), so[p])
            return carry

        lax.fori_loop(0, n_s // 2, body, 0)
        for p in range(2):
            pltpu.make_async_copy(
                outv[p], out_dst(n_s - 2 + p), so[p]).wait()

    return sc_aggregate


def kernel(q_pts, s_pts, s_feats, neighb_inds, W1, b1, W2, b2,
           gn_gamma, gn_beta, kernel_points):
    M, H = neighb_inds.shape
    assert H == 32 and s_feats.shape[1] == C

    qpw = ((M + NW - 1) // NW + 7) // 8 * 8
    mp = NW * qpw
    n_chunks_w = qpw // CHQ
    nct = NW * n_chunks_w
    n_pair = 2 * n_chunks_w

    idx = neighb_inds.astype(jnp.int32)
    idx_pad = jnp.zeros((mp, H), jnp.int32).at[:M].set(idx)
    idx2d = idx_pad.reshape(nct, 128)

    spts8 = jnp.zeros((M, 8), jnp.float32).at[:, :3].set(s_pts)
    qp16 = jnp.zeros((mp, 16), jnp.float32).at[:M, :3].set(q_pts)
    kp8 = jnp.zeros((16, 8), jnp.float32).at[:K, :3].set(kernel_points)

    sf_pad = jnp.zeros((mp, C), jnp.float32).at[:M].set(s_feats)

    xpl, ypl, zpl = _make_sc_gather_pts(nct, n_chunks_w, M)(spts8, idx2d, qp16)

    bm = 1024 if mp % 1024 == 0 else qpw
    n_blocks = mp // bm
    cw_pad, muinv = pl.pallas_call(
        functools.partial(_mlp_body, M, n_blocks, bm),
        grid=(n_blocks,),
        in_specs=[
            pl.BlockSpec((bm, C), lambda i: (i, 0)),
            pl.BlockSpec((C, C), lambda i: (0, 0)),
            pl.BlockSpec((1, C), lambda i: (0, 0)),
            pl.BlockSpec((C, KCPG), lambda i: (0, 0)),
            pl.BlockSpec((1, KCPG), lambda i: (0, 0)),
        ],
        out_specs=[
            pl.BlockSpec((bm, KCPG), lambda i: (i, 0)),
            pl.BlockSpec((8, KCPG), lambda i: (0, 0)),
        ],
        out_shape=[
            jax.ShapeDtypeStruct((mp, KCPG), jnp.float32),
            jax.ShapeDtypeStruct((8, KCPG), jnp.float32),
        ],
        scratch_shapes=[pltpu.VMEM((8, KCPG), jnp.float32)],
    )(sf_pad, W1, b1.reshape(1, C), W2, b2.reshape(1, KCPG))

    cb = 320 if nct % 320 == 0 else n_chunks_w
    cwn, bestk_pl, infl_pl = pl.pallas_call(
        functools.partial(_weights_body, cb),
        grid=(nct // cb,),
        in_specs=[
            pl.BlockSpec((cb * CHQ, KCPG), lambda i: (i, 0)),
            pl.BlockSpec((8, KCPG), lambda i: (0, 0)),
            pl.BlockSpec((1, KCPG), lambda i: (0, 0)),
            pl.BlockSpec((1, KCPG), lambda i: (0, 0)),
            pl.BlockSpec((cb, 128), lambda i: (i, 0)),
            pl.BlockSpec((cb, 128), lambda i: (i, 0)),
            pl.BlockSpec((cb, 128), lambda i: (i, 0)),
            pl.BlockSpec((16, 8), lambda i: (0, 0)),
        ],
        out_specs=[
            pl.BlockSpec((cb * CHQ, KCPG), lambda i: (i, 0)),
            pl.BlockSpec((cb, 128), lambda i: (i, 0)),
            pl.BlockSpec((cb, 128), lambda i: (i, 0)),
        ],
        out_shape=[
            jax.ShapeDtypeStruct((mp, KCPG), jnp.float32),
            jax.ShapeDtypeStruct((nct, 128), jnp.int32),
            jax.ShapeDtypeStruct((nct, 128), jnp.float32),
        ],
    )(cw_pad, muinv, gn_gamma.reshape(1, KCPG), gn_beta.reshape(1, KCPG),
      xpl, ypl, zpl, kp8)

    sf16 = s_feats.astype(jnp.bfloat16).reshape(M, 4, 2, CPG)
    sf16 = jnp.swapaxes(sf16, 2, 3)
    featsP = jax.lax.bitcast_convert_type(sf16, jnp.int32).reshape(M, C // 2)
    out_pad = _make_sc_aggregate(mp, n_pair, M)(
        featsP, idx2d, bestk_pl, infl_pl, cwn)
    return out_pad[:M]


# speedup vs baseline: 1.0267x; 1.0267x over previous
import functools

import jax
import jax.numpy as jnp
from jax import lax
from jax.experimental import pallas as pl
from jax.experimental.pallas import tpu as pltpu
from jax.experimental.pallas import tpu_sc as plsc

C = 128
K = 15
CPG = 16
GROUPS = 8
SIGMA = 2.0
EPS = 1e-5

NC = 2
NS = 16
NW = NC * NS
CHQ = 4
KCPG = K * CPG
CHALF = C // 2


def _make_sc_gather_pts(n_chunks_total, n_chunks_w, m_tab):
    mesh = plsc.VectorSubcoreMesh(core_axis_name="c", subcore_axis_name="s")

    @functools.partial(
        pl.kernel,
        mesh=mesh,
        out_type=[jax.ShapeDtypeStruct((n_chunks_total, 128), jnp.float32)
                  for _ in range(3)],
        compiler_params=pltpu.CompilerParams(
            use_tc_tiling_on_sc=False, needs_layout_passes=False),
        scratch_types=[
            pltpu.VMEM((m_tab, 8), jnp.float32),
            pltpu.VMEM((n_chunks_w, 128), jnp.int32),
            pltpu.VMEM((CHQ * n_chunks_w, 16), jnp.float32),
            pltpu.VMEM((3, 128), jnp.float32),
            pltpu.VMEM((3, 128), jnp.float32),
            pltpu.SemaphoreType.DMA,
            pltpu.SemaphoreType.DMA,
        ],
    )
    def sc_gather_pts(spts_hbm, idx_hbm, qp_hbm, x_hbm, y_hbm, z_hbm,
                      spts_v, idx_f, qp_f, pts0, pts1, so0, so1):
        wid = lax.axis_index("s") * NC + lax.axis_index("c")
        pts = (pts0, pts1)
        so = (so0, so1)
        planes = (x_hbm, y_hbm, z_hbm)

        pltpu.sync_copy(spts_hbm, spts_v)
        pltpu.sync_copy(idx_hbm.at[pl.ds(wid * n_chunks_w, n_chunks_w)], idx_f)
        pltpu.sync_copy(
            qp_hbm.at[pl.ds(wid * n_chunks_w * CHQ, n_chunks_w * CHQ)], qp_f)

        def start_out(c, p):
            gc = wid * n_chunks_w + c
            for coord in range(3):
                pltpu.async_copy(pts[p].at[coord], planes[coord].at[gc], so[p])

        def wait_out(c, p):
            gc = wid * n_chunks_w + c
            for coord in range(3):
                pltpu.make_async_copy(
                    pts[p].at[coord], planes[coord].at[gc], so[p]).wait()

        def body(c2, carry):
            for p in range(2):
                c = c2 * 2 + p

                @pl.when(c >= 2)
                def _():
                    wait_out(c, p)

                for j in range(8):
                    qrow = qp_f[c * CHQ + j // 2, :]
                    idx16 = idx_f[c, pl.ds(j * 16, 16)]
                    for coord in range(3):
                        cidx = jnp.full((16,), coord, jnp.int32)
                        v = plsc.load_gather(spts_v, [idx16, cidx])
                        pts[p][coord, pl.ds(j * 16, 16)] = v - qrow[coord]
                start_out(c, p)
            return carry

        lax.fori_loop(0, n_chunks_w // 2, body, 0)
        for p in range(2):
            wait_out(n_chunks_w - 2 + p, p)

    return sc_gather_pts


def _mlp_body(m_real, n_blocks, bm, sf_ref, w1_ref, b1_ref, w2_ref, b2_ref,
              cw_ref, muinv_ref, acc_ref):
    i = pl.program_id(0)
    h = jnp.dot(sf_ref[...], w1_ref[...], preferred_element_type=jnp.float32)
    h = h + b1_ref[...]
    h = jnp.where(h >= 0.0, h, 0.1 * h)
    cw = jnp.dot(h, w2_ref[...], preferred_element_type=jnp.float32)
    cw = cw + b2_ref[...]
    cw_ref[...] = cw

    row = i * bm + lax.broadcasted_iota(jnp.int32, (bm, 1), 0)
    mask = row < m_real
    cw_m = jnp.where(mask, cw, 0.0)
    s1 = jnp.sum(cw_m, axis=0, keepdims=True)
    s2 = jnp.sum(cw_m * cw_m, axis=0, keepdims=True)

    @pl.when(i == 0)
    def _():
        acc_ref[...] = jnp.zeros_like(acc_ref)

    acc_ref[0:1, :] += s1
    acc_ref[1:2, :] += s2

    @pl.when(i == n_blocks - 1)
    def _():
        inv_n = 1.0 / (CPG * m_real)
        for k in range(K):
            sl = slice(k * CPG, (k + 1) * CPG)
            gs1 = jnp.sum(acc_ref[0:1, sl])
            gs2 = jnp.sum(acc_ref[1:2, sl])
            mu = gs1 * inv_n
            var = gs2 * inv_n - mu * mu
            inv_sd = lax.rsqrt(var + EPS)
            muinv_ref[0:1, sl] = jnp.full((1, CPG), mu, jnp.float32)
            muinv_ref[1:2, sl] = jnp.full((1, CPG), inv_sd, jnp.float32)


def _weights_body(cb, cw_ref, muinv_ref, gam_ref, bet_ref,
                  x_ref, y_ref, z_ref, kp_ref, cwn_ref, bestk_ref, infl_ref):
    x = x_ref[...]
    y = y_ref[...]
    z = z_ref[...]

    base = x * x + y * y + z * z
    best = jnp.full((cb, 128), 1e30, dtype=jnp.float32)
    bestk = jnp.zeros((cb, 128), dtype=jnp.int32)
    for k in range(K):
        kx = kp_ref[k : k + 1, 0:1]
        ky = kp_ref[k : k + 1, 1:2]
        kz = kp_ref[k : k + 1, 2:3]
        c_k = kx * kx + ky * ky + kz * kz
        sq = c_k - 2.0 * (x * kx + y * ky + z * kz)
        better = sq < best
        best = jnp.where(better, sq, best)
        bestk = jnp.where(better, k, bestk)

    nn_sq = jnp.maximum(best + base, 0.0)
    infl_ref[...] = jnp.maximum(1.0 - jnp.sqrt(nn_sq) / SIGMA, 0.0)
    bestk_ref[...] = bestk
    cwn_ref[...] = ((cw_ref[...] - muinv_ref[0:1, :]) * muinv_ref[1:2, :]
                    * gam_ref[...] + bet_ref[...])


def _make_sc_aggregate(mp, n_pair, m_tab):
    mesh = plsc.VectorSubcoreMesh(core_axis_name="c", subcore_axis_name="s")

    n_w = n_pair // 2
    n_s = n_w // 2
    SQ = 2 * CHQ

    @functools.partial(
        pl.kernel,
        mesh=mesh,
        out_type=jax.ShapeDtypeStruct((mp, C), jnp.float32),
        compiler_params=pltpu.CompilerParams(
            use_tc_tiling_on_sc=False, needs_layout_passes=False),
        scratch_types=[
            pltpu.VMEM_SHARED((m_tab, C // 2), jnp.int32),
            pltpu.VMEM((n_w, 128), jnp.int32),
            pltpu.VMEM((n_w, 128), jnp.int32),
            pltpu.VMEM((n_w, 128), jnp.float32),
            pltpu.VMEM((SQ, KCPG), jnp.float32),
            pltpu.VMEM((SQ, KCPG), jnp.float32),
            pltpu.VMEM((256 * CPG,), jnp.float32),
            pltpu.VMEM((256, C // 2), jnp.int32),
            pltpu.VMEM((256, C // 2), jnp.int32),
            pltpu.VMEM((SQ, C), jnp.float32),
            pltpu.VMEM((SQ, C), jnp.float32),
            pltpu.SemaphoreType.DMA,
            pltpu.SemaphoreType.DMA,
            pltpu.SemaphoreType.DMA,
            pltpu.SemaphoreType.DMA,
            pltpu.SemaphoreType.DMA,
            pltpu.SemaphoreType.DMA,
        ],
    )
    def sc_aggregate(feats_hbm, idx_hbm, bk_hbm, infl_hbm, cwn_hbm, out_hbm,
                     feats_spm, idx_f, bk_f, in_f, cw0, cw1, wf_v, rows0, rows1,
                     out0, out1, sg0, sg1, sc0, sc1, so0, so1):
        core = lax.axis_index("c")
        tile = lax.axis_index("s")
        wid = tile * NC + core

        @pl.when(tile == 0)
        def _():
            pltpu.sync_copy(feats_hbm, feats_spm)

        plsc.subcore_barrier()
        lane16 = jnp.arange(16, dtype=jnp.int32) * CPG
        rows = (rows0, rows1)
        cw = (cw0, cw1)
        outv = (out0, out1)
        sg = (sg0, sg1)
        sc = (sc0, sc1)
        so = (so0, so1)

        base = wid * n_w

        pltpu.sync_copy(idx_hbm.at[pl.ds(base, n_w)], idx_f)
        pltpu.sync_copy(bk_hbm.at[pl.ds(base, n_w)], bk_f)
        pltpu.sync_copy(infl_hbm.at[pl.ds(base, n_w)], in_f)

        def start_g(s, p):
            pltpu.async_copy(feats_spm.at[idx_f.at[2 * s]],
                             rows[p].at[pl.ds(0, 128)], sg[p])
            pltpu.async_copy(feats_spm.at[idx_f.at[2 * s + 1]],
                             rows[p].at[pl.ds(128, 128)], sg[p])
            pltpu.async_copy(cwn_hbm.at[pl.ds((base + 2 * s) * CHQ, SQ)],
                             cw[p], sc[p])

        def wait_g(s, p):
            pltpu.make_async_copy(feats_spm.at[idx_f.at[2 * s]],
                                  rows[p].at[pl.ds(0, 128)], sg[p]).wait()
            pltpu.make_async_copy(feats_spm.at[idx_f.at[2 * s + 1]],
                                  rows[p].at[pl.ds(128, 128)], sg[p]).wait()
            pltpu.make_async_copy(cwn_hbm.at[pl.ds((base + 2 * s) * CHQ, SQ)],
                                  cw[p], sc[p]).wait()

        def out_dst(s):
            return out_hbm.at[pl.ds((base + 2 * s) * CHQ, SQ)]

        start_g(0, 0)
        start_g(1, 1)

        def body(s2, carry):
            for p in range(2):
                s = s2 * 2 + p
                wait_g(s, p)

                @pl.when(s >= 2)
                def _():
                    pltpu.make_async_copy(outv[p], out_dst(s), so[p]).wait()

                def wbody(j, carry2):
                    r = 2 * s + j // 8
                    sl = (j % 8) * 16
                    k16 = bk_f[r, pl.ds(sl, 16)]
                    f16 = in_f[r, pl.ds(sl, 16)]
                    q16 = jnp.full((16,), 0, jnp.int32) + j // 2
                    src = k16 * CPG
                    dst = j * 256 + lane16
                    for cc in range(CPG):
                        wc = plsc.load_gather(cw[p], [q16, src + cc]) * f16
                        plsc.store_scatter(wf_v, [dst + cc], wc)
                    return carry2

                lax.fori_loop(0, 16, wbody, 0)

                for q in range(SQ):
                    def hbody(h2, accs):
                        res = list(accs)
                        for dh in range(2):
                            e = q * 32 + h2 * 2 + dh
                            w = wf_v[pl.ds(e * CPG, CPG)]
                            for L in range(4):
                                v = rows[p][e, pl.ds(L * 16, 16)]
                                vb = plsc.bitcast(v, jnp.bfloat16)
                                a, b = plsc.unpack(
                                    vb, format=plsc.PackFormat.INTERLEAVED)
                                res[2 * L] = res[2 * L] + a * w
                                res[2 * L + 1] = res[2 * L + 1] + b * w
                        return tuple(res)
                    accs = lax.fori_loop(
                        0, 16, hbody,
                        tuple(jnp.zeros((CPG,), jnp.float32)
                              for _ in range(GROUPS)),
                    )
                    for g in range(GROUPS):
                        outv[p][q, pl.ds(g * CPG, CPG)] = accs[g]

                @pl.when(s + 2 < n_s)
                def _():
                    start_g(s + 2, p)

                pltpu.async_copy(outv[p], out_dst(s), so[p])
            return carry

        lax.fori_loop(0, n_s // 2, body, 0)
        for p in range(2):
            pltpu.make_async_copy(
                outv[p], out_dst(n_s - 2 + p), so[p]).wait()

    return sc_aggregate


def kernel(q_pts, s_pts, s_feats, neighb_inds, W1, b1, W2, b2,
           gn_gamma, gn_beta, kernel_points):
    M, H = neighb_inds.shape
    assert H == 32 and s_feats.shape[1] == C

    qpw = ((M + NW - 1) // NW + 7) // 8 * 8
    mp = NW * qpw
    n_chunks_w = qpw // CHQ
    nct = NW * n_chunks_w
    n_pair = 2 * n_chunks_w

    idx = neighb_inds.astype(jnp.int32)
    idx_pad = jnp.zeros((mp, H), jnp.int32).at[:M].set(idx)
    idx2d = idx_pad.reshape(nct, 128)

    spts8 = jnp.zeros((M, 8), jnp.float32).at[:, :3].set(s_pts)
    qp16 = jnp.zeros((mp, 16), jnp.float32).at[:M, :3].set(q_pts)
    kp8 = jnp.zeros((16, 8), jnp.float32).at[:K, :3].set(kernel_points)

    sf_pad = jnp.zeros((mp, C), jnp.float32).at[:M].set(s_feats)

    xpl, ypl, zpl = _make_sc_gather_pts(nct, n_chunks_w, M)(spts8, idx2d, qp16)

    bm = 1024 if mp % 1024 == 0 else qpw
    n_blocks = mp // bm
    cw_pad, muinv = pl.pallas_call(
        functools.partial(_mlp_body, M, n_blocks, bm),
        grid=(n_blocks,),
        in_specs=[
            pl.BlockSpec((bm, C), lambda i: (i, 0)),
            pl.BlockSpec((C, C), lambda i: (0, 0)),
            pl.BlockSpec((1, C), lambda i: (0, 0)),
            pl.BlockSpec((C, KCPG), lambda i: (0, 0)),
            pl.BlockSpec((1, KCPG), lambda i: (0, 0)),
        ],
        out_specs=[
            pl.BlockSpec((bm, KCPG), lambda i: (i, 0)),
            pl.BlockSpec((8, KCPG), lambda i: (0, 0)),
        ],
        out_shape=[
            jax.ShapeDtypeStruct((mp, KCPG), jnp.float32),
            jax.ShapeDtypeStruct((8, KCPG), jnp.float32),
        ],
        scratch_shapes=[pltpu.VMEM((8, KCPG), jnp.float32)],
    )(sf_pad, W1, b1.reshape(1, C), W2, b2.reshape(1, KCPG))

    cb = 320 if nct % 320 == 0 else n_chunks_w
    cwn, bestk_pl, infl_pl = pl.pallas_call(
        functools.partial(_weights_body, cb),
        grid=(nct // cb,),
        in_specs=[
            pl.BlockSpec((cb * CHQ, KCPG), lambda i: (i, 0)),
            pl.BlockSpec((8, KCPG), lambda i: (0, 0)),
            pl.BlockSpec((1, KCPG), lambda i: (0, 0)),
            pl.BlockSpec((1, KCPG), lambda i: (0, 0)),
            pl.BlockSpec((cb, 128), lambda i: (i, 0)),
            pl.BlockSpec((cb, 128), lambda i: (i, 0)),
            pl.BlockSpec((cb, 128), lambda i: (i, 0)),
            pl.BlockSpec((16, 8), lambda i: (0, 0)),
        ],
        out_specs=[
            pl.BlockSpec((cb * CHQ, KCPG), lambda i: (i, 0)),
            pl.BlockSpec((cb, 128), lambda i: (i, 0)),
            pl.BlockSpec((cb, 128), lambda i: (i, 0)),
        ],
        out_shape=[
            jax.ShapeDtypeStruct((mp, KCPG), jnp.float32),
            jax.ShapeDtypeStruct((nct, 128), jnp.int32),
            jax.ShapeDtypeStruct((nct, 128), jnp.float32),
        ],
    )(cw_pad, muinv, gn_gamma.reshape(1, KCPG), gn_beta.reshape(1, KCPG),
      xpl, ypl, zpl, kp8)

    sf16 = s_feats.astype(jnp.bfloat16).reshape(M, 4, 2, CPG)
    sf16 = jnp.swapaxes(sf16, 2, 3)
    featsP = jax.lax.bitcast_convert_type(sf16, jnp.int32).reshape(M, C // 2)
    out_pad = _make_sc_aggregate(mp, n_pair, M)(
        featsP, idx2d, bestk_pl, infl_pl, cwn)
    return out_pad[:M]
